# baseline (speedup 1.0000x reference)
import functools

import jax
import jax.numpy as jnp
from jax import lax
from jax.experimental import pallas as pl
from jax.experimental.pallas import tpu as pltpu
from jax.experimental.pallas import tpu_sc as plsc

B = 4
S = 2048
D = 1024
H = D // 2
MD0, MD1, MD2 = 2050, 64, 64
OFFSET = 2

NC = 2
NS = 16
ROWS = B * S
ROWS_PER_T = ROWS // NS
CHUNK = 32
NCHUNK = ROWS_PER_T // CHUNK


def _shift_right(x, k, fill):
    pad = jnp.full((B, k), fill, dtype=x.dtype)
    return jnp.concatenate([pad, x[:, : S - k]], axis=1)


def _cumsum(x):
    k = 1
    while k < S:
        x = x + _shift_right(x, k, 0)
        k *= 2
    return x


def _cummax(x, fill):
    k = 1
    while k < S:
        x = jnp.maximum(x, _shift_right(x, k, fill))
        k *= 2
    return x


def _maps_body(ids_ref, m0_ref, m1_ref, m2_ref, cnt_ref):
    tok = ids_ref[...]
    c1 = jnp.logical_and(tok >= 5, tok <= 8)
    c2 = jnp.logical_and(tok >= 9, tok <= 10)
    i32 = jnp.int32
    t = lax.broadcasted_iota(i32, (B, S), 1)
    e = (tok == 1).astype(i32)
    done = _cumsum(e) > 0
    s1 = _cumsum(c1.astype(i32))
    cc2 = _cumsum(c2.astype(i32))
    lastreset = _cummax(jnp.where(jnp.logical_or(c1, c2), t, -1), -1)
    n0raw = jnp.where(lastreset >= 0, t - lastreset, t + 1 + OFFSET)
    ov0 = n0raw == MD0
    n0 = jnp.where(ov0, 0, n0raw)
    v = _cummax(jnp.where(c2, s1, 0), 0)
    n1c = s1 - v
    wrap1 = jnp.logical_and(c1, (n1c & 63) == 0)
    w = _cumsum(wrap1.astype(i32))
    n1 = (n1c & 63) + ov0.astype(i32)
    n2 = (cc2 + w) & 63
    m0_ref[...] = jnp.where(done, MD0 - 1, n0)
    m1_ref[...] = jnp.where(done, MD1 - 1, n1)
    m2_ref[...] = jnp.where(done, MD2 - 1, n2)
    p = jnp.sum(jnp.logical_not(done).astype(i32), axis=1, keepdims=True)
    sel = t == (p - 1)
    f0 = jnp.sum(jnp.where(sel, n0, 0), axis=1, keepdims=True)
    f1 = jnp.sum(jnp.where(sel, n1, 0), axis=1, keepdims=True)
    f2 = jnp.sum(jnp.where(sel, n2, 0), axis=1, keepdims=True)
    f0 = jnp.where(p == 0, OFFSET, f0)
    f1 = jnp.where(p == 0, 0, f1)
    f2 = jnp.where(p == 0, 0, f2)
    col = lax.broadcasted_iota(i32, (B, 128), 1)
    cnt = jnp.where(col == 0, f0, jnp.where(col == 1, f1, jnp.where(col == 2, f2, 0)))
    cnt_ref[...] = cnt


def _compute_maps(input_ids, interpret=False):
    out = pl.pallas_call(
        _maps_body,
        out_shape=[
            jax.ShapeDtypeStruct((B, S), jnp.int32),
            jax.ShapeDtypeStruct((B, S), jnp.int32),
            jax.ShapeDtypeStruct((B, S), jnp.int32),
            jax.ShapeDtypeStruct((B, 128), jnp.int32),
        ],
        interpret=interpret,
    )(input_ids)
    return out


def _gather_body(m0h, m1h, m2h, e0h, e1h, e2h, outh,
                 i0, i1, i2, ba, bb, loc1, loc2,
                 sa0, sa1, sb0, sb1, ssa0, ssa1, ssb0, ssb1, si):
    c = lax.axis_index("c")
    s = lax.axis_index("s")
    base = s * ROWS_PER_T

    ci0 = pltpu.async_copy(m0h.at[pl.ds(base, ROWS_PER_T)], i0, si)
    pltpu.sync_copy(e1h.at[pl.ds(c * MD1, MD1)], loc1)
    pltpu.sync_copy(e2h.at[pl.ds(c * MD2, MD2)], loc2)
    ci0.wait()
    ci1 = pltpu.async_copy(m1h.at[pl.ds(base, ROWS_PER_T)], i1, si)
    ci2 = pltpu.async_copy(m2h.at[pl.ds(base, ROWS_PER_T)], i2, si)
    off = c * MD0

    @plsc.parallel_loop(0, ROWS_PER_T // 16, 1, unroll=4)
    def rebase(h):
        sl = pl.ds(h * 16, 16)
        i0.at[sl][...] = i0.at[sl][...] + off

    ci1.wait()
    ci2.wait()

    def do_adds(buf, ci):
        m1s, m2s = [], []
        for h in range(CHUNK // 16):
            sl = pl.ds(ci * CHUNK + h * 16, 16)
            i1v = i1.at[sl][...]
            i2v = i2.at[sl][...]
            for r in range(16):
                m1s.append(i1v[r])
                m2s.append(i2v[r])

        @plsc.parallel_loop(0, H // 16, 1)
        def col_body(cb):
            csl = pl.ds(cb * 16, 16)
            for row in range(CHUNK):
                v = loc1.at[m1s[row], csl][...] + loc2.at[m2s[row], csl][...]
                plsc.addupdate(buf.at[row, csl], v)

    HC = CHUNK // 2

    def chunk_flags(ci):
        v0 = i0.at[pl.ds(ci * CHUNK, 16)][...]
        v1 = i0.at[pl.ds(ci * CHUNK + 16, 16)][...]
        idx0 = v0[0]
        io = lax.iota(jnp.int32, 16)
        ramp = jnp.logical_and(jnp.all(v0 == idx0 + io),
                               jnp.all(v1 == idx0 + 16 + io))
        const = jnp.logical_and(jnp.all(v0 == idx0), jnp.all(v1 == idx0))
        return idx0, ramp, const

    def gather_streams(buf, sems, ci, start):
        idx0, ramp, const = chunk_flags(ci)
        othr = jnp.logical_not(jnp.logical_or(ramp, const))

        def go(s0, d0, s1, d1):
            c0 = pltpu.make_async_copy(s0, d0, sems[0])
            c1 = pltpu.make_async_copy(s1, d1, sems[1])
            if start:
                c0.start()
                c1.start()
            else:
                c0.wait()
                c1.wait()

        @pl.when(ramp)
        def _():
            go(e0h.at[pl.ds(idx0, HC)], buf.at[pl.ds(0, HC)],
               e0h.at[pl.ds(idx0 + HC, HC)], buf.at[pl.ds(HC, HC)])

        @pl.when(const)
        def _():
            go(e0h.at[pl.ds(idx0, 1)], buf.at[pl.ds(0, 1)],
               e0h.at[pl.ds(idx0, 1)], buf.at[pl.ds(HC, 1)])

        @pl.when(othr)
        def _():
            go(e0h.at[i0.at[pl.ds(ci * CHUNK, HC)]], buf.at[pl.ds(0, HC)],
               e0h.at[i0.at[pl.ds(ci * CHUNK + HC, HC)]],
               buf.at[pl.ds(HC, HC)])

        if not start:
            @pl.when(const)
            def _():
                @plsc.parallel_loop(0, H // 16, 1)
                def rep(cb):
                    csl = pl.ds(cb * 16, 16)
                    r0 = buf.at[0, csl][...]
                    r1 = buf.at[HC, csl][...]
                    for rr in range(1, HC):
                        buf.at[rr, csl][...] = r0
                        buf.at[HC + rr, csl][...] = r1

    def gather_start(buf, sems, ci):
        gather_streams(buf, sems, ci, True)

    def gather_wait(buf, sems, ci):
        gather_streams(buf, sems, ci, False)

    def out_ref(ci, lo, n):
        return outh.at[pl.ds(base + ci * CHUNK + lo, n), c]

    def scatter_start(buf, sems, ci):
        pltpu.async_copy(buf.at[pl.ds(0, HC)], out_ref(ci, 0, HC), sems[0])
        pltpu.async_copy(buf.at[pl.ds(HC, HC)], out_ref(ci, HC, HC), sems[1])

    def scatter_wait(buf, sems, ci):
        pltpu.make_async_copy(buf.at[pl.ds(0, HC)], out_ref(ci, 0, HC),
                              sems[0]).wait()
        pltpu.make_async_copy(buf.at[pl.ds(HC, HC)], out_ref(ci, HC, HC),
                              sems[1]).wait()

    gather_start(ba, (sa0, sa1), 0)

    def step_pair(gi, carry):
        ci = gi * 2

        def one(buf, gsems, ssems, obuf, ogsems, ossems, ci):
            nxt = ci + 1

            @pl.when(jnp.logical_and(nxt < NCHUNK, ci >= 1))
            def _():
                scatter_wait(obuf, ossems, ci - 1)

            @pl.when(nxt < NCHUNK)
            def _():
                gather_start(obuf, ogsems, nxt)

            gather_wait(buf, gsems, ci)
            do_adds(buf, ci)
            scatter_start(buf, ssems, ci)

        one(ba, (sa0, sa1), (ssa0, ssa1), bb, (sb0, sb1), (ssb0, ssb1), ci)
        one(bb, (sb0, sb1), (ssb0, ssb1), ba, (sa0, sa1), (ssa0, ssa1), ci + 1)
        return carry

    lax.fori_loop(0, NCHUNK // 2, step_pair, 0, unroll=False)
    scatter_wait(ba, (ssa0, ssa1), NCHUNK - 2)
    scatter_wait(bb, (ssb0, ssb1), NCHUNK - 1)


def _gather_sum(m0f, m1f, m2f, e0s, e1s, e2s):
    mesh = plsc.VectorSubcoreMesh(
        core_axis_name="c", subcore_axis_name="s",
        num_cores=NC, num_subcores=NS)
    kern = pl.kernel(
        _gather_body,
        out_type=jax.ShapeDtypeStruct((ROWS, NC, H), jnp.float32),
        mesh=mesh,
        compiler_params=pltpu.CompilerParams(needs_layout_passes=False, use_tc_tiling_on_sc=False),
        scratch_types=[
            pltpu.VMEM((ROWS_PER_T,), jnp.int32),
            pltpu.VMEM((ROWS_PER_T,), jnp.int32),
            pltpu.VMEM((ROWS_PER_T,), jnp.int32),
            pltpu.VMEM((CHUNK, H), jnp.float32),
            pltpu.VMEM((CHUNK, H), jnp.float32),
            pltpu.VMEM((MD1, H), jnp.float32),
            pltpu.VMEM((MD2, H), jnp.float32),
            pltpu.SemaphoreType.DMA,
            pltpu.SemaphoreType.DMA,
            pltpu.SemaphoreType.DMA,
            pltpu.SemaphoreType.DMA,
            pltpu.SemaphoreType.DMA,
            pltpu.SemaphoreType.DMA,
            pltpu.SemaphoreType.DMA,
            pltpu.SemaphoreType.DMA,
            pltpu.SemaphoreType.DMA,
        ],
    )
    return kern(m0f, m1f, m2f, e0s, e1s, e2s)


@jax.jit
def kernel(input_ids, emb0, emb1, emb2):
    m0, m1, m2, cnt = _compute_maps(input_ids)
    counters = cnt[:, :3]
    e0s = jnp.concatenate([emb0[:, :H], emb0[:, H:]], axis=0)
    e1s = jnp.concatenate([emb1[:, :H], emb1[:, H:]], axis=0)
    e2s = jnp.concatenate([emb2[:, :H], emb2[:, H:]], axis=0)
    out = _gather_sum(m0.reshape(ROWS), m1.reshape(ROWS), m2.reshape(ROWS),
                      e0s, e1s, e2s)
    return out.reshape(B, S, D), counters


# speedup vs baseline: 1.0967x; 1.0967x over previous
import functools

import jax
import jax.numpy as jnp
from jax import lax
from jax.experimental import pallas as pl
from jax.experimental.pallas import tpu as pltpu
from jax.experimental.pallas import tpu_sc as plsc

B = 4
S = 2048
D = 1024
H = D // 2
MD0, MD1, MD2 = 2050, 64, 64
OFFSET = 2

NC = 2
NS = 16
ROWS = B * S
ROWS_PER_T = ROWS // NS
CHUNK = 16
NCHUNK = ROWS_PER_T // CHUNK
NBUF = 4


def _shift_right(x, k, fill):
    pad = jnp.full((B, k), fill, dtype=x.dtype)
    return jnp.concatenate([pad, x[:, : S - k]], axis=1)


def _cumsum(x):
    k = 1
    while k < S:
        x = x + _shift_right(x, k, 0)
        k *= 2
    return x


def _cummax(x, fill):
    k = 1
    while k < S:
        x = jnp.maximum(x, _shift_right(x, k, fill))
        k *= 2
    return x


def _maps_body(ids_ref, m0_ref, m1_ref, m2_ref, cnt_ref):
    tok = ids_ref[...]
    c1 = jnp.logical_and(tok >= 5, tok <= 8)
    c2 = jnp.logical_and(tok >= 9, tok <= 10)
    i32 = jnp.int32
    t = lax.broadcasted_iota(i32, (B, S), 1)
    e = (tok == 1).astype(i32)
    done = _cumsum(e) > 0
    s1 = _cumsum(c1.astype(i32))
    cc2 = _cumsum(c2.astype(i32))
    lastreset = _cummax(jnp.where(jnp.logical_or(c1, c2), t, -1), -1)
    n0raw = jnp.where(lastreset >= 0, t - lastreset, t + 1 + OFFSET)
    ov0 = n0raw == MD0
    n0 = jnp.where(ov0, 0, n0raw)
    v = _cummax(jnp.where(c2, s1, 0), 0)
    n1c = s1 - v
    wrap1 = jnp.logical_and(c1, (n1c & 63) == 0)
    w = _cumsum(wrap1.astype(i32))
    n1 = (n1c & 63) + ov0.astype(i32)
    n2 = (cc2 + w) & 63
    m0_ref[...] = jnp.where(done, MD0 - 1, n0)
    m1_ref[...] = jnp.where(done, MD1 - 1, n1)
    m2_ref[...] = jnp.where(done, MD2 - 1, n2)
    p = jnp.sum(jnp.logical_not(done).astype(i32), axis=1, keepdims=True)
    sel = t == (p - 1)
    f0 = jnp.sum(jnp.where(sel, n0, 0), axis=1, keepdims=True)
    f1 = jnp.sum(jnp.where(sel, n1, 0), axis=1, keepdims=True)
    f2 = jnp.sum(jnp.where(sel, n2, 0), axis=1, keepdims=True)
    f0 = jnp.where(p == 0, OFFSET, f0)
    f1 = jnp.where(p == 0, 0, f1)
    f2 = jnp.where(p == 0, 0, f2)
    col = lax.broadcasted_iota(i32, (B, 128), 1)
    cnt = jnp.where(col == 0, f0, jnp.where(col == 1, f1, jnp.where(col == 2, f2, 0)))
    cnt_ref[...] = cnt


def _compute_maps(input_ids, interpret=False):
    out = pl.pallas_call(
        _maps_body,
        out_shape=[
            jax.ShapeDtypeStruct((B, S), jnp.int32),
            jax.ShapeDtypeStruct((B, S), jnp.int32),
            jax.ShapeDtypeStruct((B, S), jnp.int32),
            jax.ShapeDtypeStruct((B, 128), jnp.int32),
        ],
        interpret=interpret,
    )(input_ids)
    return out


def _gather_body(m0h, m1h, m2h, e0h, e1h, e2h, outh,
                 i0, i1, i2, ba, bb, bc, bd, loc1, loc2,
                 sa0, sa1, sb0, sb1, ssa0, ssa1, ssb0, ssb1, si):
    c = lax.axis_index("c")
    s = lax.axis_index("s")
    base = s * ROWS_PER_T

    ci0 = pltpu.async_copy(m0h.at[pl.ds(base, ROWS_PER_T)], i0, si)
    pltpu.sync_copy(e1h.at[pl.ds(c * MD1, MD1)], loc1)
    pltpu.sync_copy(e2h.at[pl.ds(c * MD2, MD2)], loc2)
    ci0.wait()
    ci1 = pltpu.async_copy(m1h.at[pl.ds(base, ROWS_PER_T)], i1, si)
    ci2 = pltpu.async_copy(m2h.at[pl.ds(base, ROWS_PER_T)], i2, si)
    off = c * MD0

    @plsc.parallel_loop(0, ROWS_PER_T // 16, 1, unroll=4)
    def rebase(h):
        sl = pl.ds(h * 16, 16)
        i0.at[sl][...] = i0.at[sl][...] + off

    ci1.wait()
    ci2.wait()

    def do_adds(buf, ci):
        m1s, m2s = [], []
        for h in range(CHUNK // 16):
            sl = pl.ds(ci * CHUNK + h * 16, 16)
            i1v = i1.at[sl][...]
            i2v = i2.at[sl][...]
            for r in range(16):
                m1s.append(i1v[r])
                m2s.append(i2v[r])

        @plsc.parallel_loop(0, H // 16, 1)
        def col_body(cb):
            csl = pl.ds(cb * 16, 16)
            for row in range(CHUNK):
                v = loc1.at[m1s[row], csl][...] + loc2.at[m2s[row], csl][...]
                plsc.addupdate(buf.at[row, csl], v)

    bufs = (ba, bb, bc, bd)
    gsems = (sa0, sa1, sb0, sb1)
    ssems = (ssa0, ssa1, ssb0, ssb1)

    def gather_start(k, ci):
        pltpu.async_copy(e0h.at[i0.at[pl.ds(ci * CHUNK, CHUNK)]],
                         bufs[k], gsems[k])

    def gather_wait(k, ci):
        pltpu.make_async_copy(e0h.at[i0.at[pl.ds(ci * CHUNK, CHUNK)]],
                              bufs[k], gsems[k]).wait()

    def out_ref(ci):
        return outh.at[pl.ds(base + ci * CHUNK, CHUNK), c]

    def scatter_start(k, ci):
        pltpu.async_copy(bufs[k], out_ref(ci), ssems[k])

    def scatter_wait(k, ci):
        pltpu.make_async_copy(bufs[k], out_ref(ci), ssems[k]).wait()

    gather_start(0, 0)
    gather_start(1, 1)

    def step_quad(g, carry):
        ci0 = g * NBUF

        def one(k, ci):
            nk = (k + 2) % NBUF

            @pl.when(jnp.logical_and(ci + 2 < NCHUNK, ci >= 2))
            def _():
                scatter_wait(nk, ci - 2)

            @pl.when(ci + 2 < NCHUNK)
            def _():
                gather_start(nk, ci + 2)

            gather_wait(k, ci)
            do_adds(bufs[k], ci)
            scatter_start(k, ci)

        for k in range(NBUF):
            one(k, ci0 + k)
        return carry

    lax.fori_loop(0, NCHUNK // NBUF, step_quad, 0, unroll=False)
    for k in range(NBUF):
        scatter_wait(k, NCHUNK - NBUF + k)


def _gather_sum(m0f, m1f, m2f, e0s, e1s, e2s):
    mesh = plsc.VectorSubcoreMesh(
        core_axis_name="c", subcore_axis_name="s",
        num_cores=NC, num_subcores=NS)
    kern = pl.kernel(
        _gather_body,
        out_type=jax.ShapeDtypeStruct((ROWS, NC, H), jnp.float32),
        mesh=mesh,
        compiler_params=pltpu.CompilerParams(needs_layout_passes=False),
        scratch_types=[
            pltpu.VMEM((ROWS_PER_T,), jnp.int32),
            pltpu.VMEM((ROWS_PER_T,), jnp.int32),
            pltpu.VMEM((ROWS_PER_T,), jnp.int32),
            pltpu.VMEM((CHUNK, H), jnp.float32),
            pltpu.VMEM((CHUNK, H), jnp.float32),
            pltpu.VMEM((CHUNK, H), jnp.float32),
            pltpu.VMEM((CHUNK, H), jnp.float32),
            pltpu.VMEM((MD1, H), jnp.float32),
            pltpu.VMEM((MD2, H), jnp.float32),
            pltpu.SemaphoreType.DMA,
            pltpu.SemaphoreType.DMA,
            pltpu.SemaphoreType.DMA,
            pltpu.SemaphoreType.DMA,
            pltpu.SemaphoreType.DMA,
            pltpu.SemaphoreType.DMA,
            pltpu.SemaphoreType.DMA,
            pltpu.SemaphoreType.DMA,
            pltpu.SemaphoreType.DMA,
        ],
    )
    return kern(m0f, m1f, m2f, e0s, e1s, e2s)


@jax.jit
def kernel(input_ids, emb0, emb1, emb2):
    m0, m1, m2, cnt = _compute_maps(input_ids)
    counters = cnt[:, :3]
    e0s = jnp.concatenate([emb0[:, :H], emb0[:, H:]], axis=0)
    e1s = jnp.concatenate([emb1[:, :H], emb1[:, H:]], axis=0)
    e2s = jnp.concatenate([emb2[:, :H], emb2[:, H:]], axis=0)
    out = _gather_sum(m0.reshape(ROWS), m1.reshape(ROWS), m2.reshape(ROWS),
                      e0s, e1s, e2s)
    return out.reshape(B, S, D), counters
